# baseline (speedup 1.0000x reference)
import jax
import jax.numpy as jnp
from jax import lax
from jax.experimental import pallas as pl

_N = 256
_H = 256
_D = 8
_ROWS_F = _D + 1 + _H + 1 + _H + 1 + _D + 1
_ROWS_F4 = 6 * _D + 1 + _H + 1 + _H + 1 + _D + 1
_ROWS = 3 * _ROWS_F + _ROWS_F4


def _dot(a, b):
    return jnp.dot(a, b, preferred_element_type=jnp.float32)


def _dot_t(a, bt):
    return lax.dot_general(a, bt, (((1,), (1,)), ((), ())),
                           preferred_element_type=jnp.float32)


def _mlp_slices(p_ref, base, in_rows):
    w1 = p_ref[base:base + in_rows, :]
    o = base + in_rows
    b1 = p_ref[o:o + 1, :]
    w2 = p_ref[o + 1:o + 1 + _H, :]
    b2 = p_ref[o + 1 + _H:o + 2 + _H, :]
    w3 = p_ref[o + 2 + _H:o + 2 + 2 * _H, :]
    b3 = p_ref[o + 2 + 2 * _H:o + 3 + 2 * _H, :]
    w4t = p_ref[o + 3 + 2 * _H:o + 3 + 2 * _H + _D, :]
    b4 = p_ref[o + 3 + 2 * _H + _D:o + 4 + 2 * _H + _D, 0:_D]
    return w1, b1, w2, b2, w3, b3, w4t, b4


def _run_mlp(x, w1, b1, w2, b2, w3, b3, w4t, b4):
    h = jax.nn.relu(_dot(x, w1) + b1)
    h = jax.nn.relu(_dot(h, w2) + b2)
    h = jax.nn.relu(_dot(h, w3) + b3)
    return _dot_t(h, w4t) + b4


def _fused_kernel(x_ref, mm_ref, p_ref, out_ref):
    xc = x_ref[0]
    init = xc
    mm = mm_ref[0]
    mmc = mm[:, None]
    mmr = mm[None, :]
    prev = (mmr == mmc - 1).astype(jnp.float32)
    nxt = (mmr == mmc + 1).astype(jnp.float32)

    f1 = _mlp_slices(p_ref, 0, _D)
    f2 = _mlp_slices(p_ref, _ROWS_F, _D)
    f3 = _mlp_slices(p_ref, 2 * _ROWS_F, _D)
    w41 = p_ref[3 * _ROWS_F:3 * _ROWS_F + 6 * _D, :]
    o4 = 3 * _ROWS_F + 6 * _D
    b41 = p_ref[o4:o4 + 1, :]
    w42 = p_ref[o4 + 1:o4 + 1 + _H, :]
    b42 = p_ref[o4 + 1 + _H:o4 + 2 + _H, :]
    w43 = p_ref[o4 + 2 + _H:o4 + 2 + 2 * _H, :]
    b43 = p_ref[o4 + 2 + 2 * _H:o4 + 3 + 2 * _H, :]
    w44t = p_ref[o4 + 3 + 2 * _H:o4 + 3 + 2 * _H + _D, :]
    b44 = p_ref[o4 + 3 + 2 * _H + _D:o4 + 4 + 2 * _H + _D, 0:_D]

    _, b31, w32, b32, w33, b33, w34t, b34 = f3
    h3 = jax.nn.relu(b31)
    h3 = jax.nn.relu(_dot(h3, w32) + b32)
    h3 = jax.nn.relu(_dot(h3, w33) + b33)
    a3 = jax.nn.relu(_dot_t(h3, w34t) + b34)

    c_const = _dot(a3, w41[16:24, :]) + _dot(init, w41[40:48, :]) + b41

    for _ in range(3):
        in1 = _dot(prev, xc)
        in2 = _dot(nxt, xc)

        a1 = jax.nn.relu(_run_mlp(in1, *f1))
        a2 = jax.nn.relu(_run_mlp(in2, *f2))
        a4 = jax.nn.relu(jnp.sum(xc, axis=0, keepdims=True))

        h = (_dot(a1, w41[0:8, :]) + _dot(a2, w41[8:16, :])
             + _dot(a4, w41[24:32, :]) + _dot(xc, w41[32:40, :]) + c_const)
        h = jax.nn.relu(h)
        h = jax.nn.relu(_dot(h, w42) + b42)
        h = jax.nn.relu(_dot(h, w43) + b43)
        xc = _dot_t(h, w44t) + b44

    out_ref[0] = xc


def _pack(params):
    pieces = []
    for name in ("f1", "f2", "f3", "f4"):
        (w1, b1), (w2, b2), (w3, b3), (w4, b4) = params[name]
        pieces += [w1, b1[None, :], w2, b2[None, :], w3, b3[None, :],
                   w4.T, jnp.pad(b4, (0, _H - _D))[None, :]]
    return jnp.concatenate(pieces, axis=0)


def kernel(x, MM, PM, params):
    J, N, d = x.shape
    packed = _pack(params)
    out = pl.pallas_call(
        _fused_kernel,
        out_shape=jax.ShapeDtypeStruct((J, N, d), jnp.float32),
    )(x, MM, packed)
    return out


# speedup vs baseline: 3.8408x; 3.8408x over previous
import jax
import jax.numpy as jnp
from jax.experimental import pallas as pl
from jax.experimental.pallas import tpu as pltpu

_BIG = 8


def _dot(a, b):
    return jnp.dot(a, b, preferred_element_type=jnp.float32)


def _fused_kernel(x_ref, mm_ref,
                  w11, b11, b12, b13, w14, b14,
                  w21, b21, b22, b23, w24, b24,
                  b31, b32, b33, w34, b34,
                  w41, b41, b42, b43, w44, b44,
                  h12, h13, h22, h23, h32, h33, h42, h43,
                  out_ref,
                  v12, v13, v22, v23, v32, v33, v42, v43,
                  sems):
    hbm = (h12, h13, h22, h23, h32, h33, h42, h43)
    vmem = (v12, v13, v22, v23, v32, v33, v42, v43)
    copies = []
    for i in range(_BIG):
        cp = pltpu.make_async_copy(hbm[i], vmem[i], sems.at[i])
        cp.start()
        copies.append(cp)

    xc = x_ref[0]
    init = xc
    mm = mm_ref[0]
    mmc = mm[:, None]
    mmr = mm[None, :]
    prev = (mmr == mmc - 1).astype(jnp.float32)
    nxt = (mmr == mmc + 1).astype(jnp.float32)

    copies[4].wait()
    copies[5].wait()
    h3 = jax.nn.relu(b31[...][None, :])
    h3 = jax.nn.relu(_dot(h3, v32[...]) + b32[...])
    h3 = jax.nn.relu(_dot(h3, v33[...]) + b33[...])
    a3 = jax.nn.relu(_dot(h3, w34[...]) + b34[...])

    c_const = _dot(a3, w41[16:24, :]) + _dot(init, w41[40:48, :]) + b41[...][None, :]

    for k in range(3):
        in1 = _dot(prev, xc)
        in2 = _dot(nxt, xc)

        h = jax.nn.relu(_dot(in1, w11[...]) + b11[...])
        if k == 0:
            copies[0].wait()
        h = jax.nn.relu(_dot(h, v12[...]) + b12[...])
        if k == 0:
            copies[1].wait()
        h = jax.nn.relu(_dot(h, v13[...]) + b13[...])
        a1 = jax.nn.relu(_dot(h, w14[...]) + b14[...])

        h = jax.nn.relu(_dot(in2, w21[...]) + b21[...])
        if k == 0:
            copies[2].wait()
        h = jax.nn.relu(_dot(h, v22[...]) + b22[...])
        if k == 0:
            copies[3].wait()
        h = jax.nn.relu(_dot(h, v23[...]) + b23[...])
        a2 = jax.nn.relu(_dot(h, w24[...]) + b24[...])

        a4 = jax.nn.relu(jnp.sum(xc, axis=0, keepdims=True))

        h = (_dot(a1, w41[0:8, :]) + _dot(a2, w41[8:16, :])
             + _dot(a4, w41[24:32, :]) + _dot(xc, w41[32:40, :]) + c_const)
        h = jax.nn.relu(h)
        if k == 0:
            copies[6].wait()
        h = jax.nn.relu(_dot(h, v42[...]) + b42[...])
        if k == 0:
            copies[7].wait()
        h = jax.nn.relu(_dot(h, v43[...]) + b43[...])
        xc = _dot(h, w44[...]) + b44[...]

    out_ref[0] = xc


def kernel(x, MM, PM, params):
    J, N, d = x.shape
    (f1w1, f1b1), (f1w2, f1b2), (f1w3, f1b3), (f1w4, f1b4) = params["f1"]
    (f2w1, f2b1), (f2w2, f2b2), (f2w3, f2b3), (f2w4, f2b4) = params["f2"]
    (f3w1, f3b1), (f3w2, f3b2), (f3w3, f3b3), (f3w4, f3b4) = params["f3"]
    (f4w1, f4b1), (f4w2, f4b2), (f4w3, f4b3), (f4w4, f4b4) = params["f4"]
    small = [f1w1, f1b1, f1b2, f1b3, f1w4, f1b4,
             f2w1, f2b1, f2b2, f2b3, f2w4, f2b4,
             f3b1, f3b2, f3b3, f3w4, f3b4,
             f4w1, f4b1, f4b2, f4b3, f4w4, f4b4]
    big = [f1w2, f1w3, f2w2, f2w3, f3w2, f3w3, f4w2, f4w3]
    vm = pltpu.MemorySpace.VMEM
    in_specs = ([pl.BlockSpec(memory_space=vm)] * (2 + len(small))
                + [pl.BlockSpec(memory_space=pltpu.MemorySpace.HBM)] * _BIG)
    scratch_shapes = ([pltpu.VMEM((256, 256), jnp.float32)] * _BIG
                      + [pltpu.SemaphoreType.DMA((_BIG,))])
    out = pl.pallas_call(
        _fused_kernel,
        out_shape=jax.ShapeDtypeStruct((J, N, d), jnp.float32),
        in_specs=in_specs,
        out_specs=pl.BlockSpec(memory_space=vm),
        scratch_shapes=scratch_shapes,
    )(x, MM, *small, *big)
    return out


# speedup vs baseline: 4.1927x; 1.0916x over previous
import jax
import jax.numpy as jnp
from jax.experimental import pallas as pl
from jax.experimental.pallas import tpu as pltpu

_BIG = 8


def _dot(a, b):
    return jnp.dot(a, b, preferred_element_type=jnp.float32)


def _fused_kernel(x_ref, mm_ref,
                  w11, b11, b12, b13, w14, b14,
                  w21, b21, b22, b23, w24, b24,
                  b31, b32, b33, w34, b34,
                  w41, b41, b42, b43, w44, b44,
                  h12, h22, h13, h23, h32, h33, h42, h43,
                  out_ref,
                  v12, v22, v13, v23, v32, v33, v42, v43,
                  sems):
    hbm = (h12, h22, h13, h23, h32, h33, h42, h43)
    vmem = (v12, v22, v13, v23, v32, v33, v42, v43)
    copies = []
    for i in range(_BIG):
        cp = pltpu.make_async_copy(hbm[i], vmem[i], sems.at[i])
        cp.start()
        copies.append(cp)

    xc = x_ref[0]
    init = xc
    mm = mm_ref[0]
    mmc = mm[:, None]
    mmr = mm[None, :]
    prev = (mmr == mmc - 1).astype(jnp.float32)
    nxt = (mmr == mmc + 1).astype(jnp.float32)

    c_const = None

    for k in range(3):
        in1 = _dot(prev, xc)
        in2 = _dot(nxt, xc)

        h1 = jax.nn.relu(_dot(in1, w11[...]) + b11[...])
        h2 = jax.nn.relu(_dot(in2, w21[...]) + b21[...])
        if k == 0:
            copies[0].wait()
        h1 = jax.nn.relu(_dot(h1, v12[...]) + b12[...])
        if k == 0:
            copies[1].wait()
        h2 = jax.nn.relu(_dot(h2, v22[...]) + b22[...])
        if k == 0:
            copies[2].wait()
        h1 = jax.nn.relu(_dot(h1, v13[...]) + b13[...])
        if k == 0:
            copies[3].wait()
        h2 = jax.nn.relu(_dot(h2, v23[...]) + b23[...])
        a1 = jax.nn.relu(_dot(h1, w14[...]) + b14[...])
        a2 = jax.nn.relu(_dot(h2, w24[...]) + b24[...])

        if k == 0:
            copies[4].wait()
            copies[5].wait()
            h3 = jax.nn.relu(b31[...][None, :])
            h3 = jax.nn.relu(_dot(h3, v32[...]) + b32[...])
            h3 = jax.nn.relu(_dot(h3, v33[...]) + b33[...])
            a3 = jax.nn.relu(_dot(h3, w34[...]) + b34[...])
            c_const = (_dot(a3, w41[16:24, :]) + _dot(init, w41[40:48, :])
                       + b41[...][None, :])

        a4 = jax.nn.relu(jnp.sum(xc, axis=0, keepdims=True))

        h = (_dot(a1, w41[0:8, :]) + _dot(a2, w41[8:16, :])
             + _dot(a4, w41[24:32, :]) + _dot(xc, w41[32:40, :]) + c_const)
        h = jax.nn.relu(h)
        if k == 0:
            copies[6].wait()
        h = jax.nn.relu(_dot(h, v42[...]) + b42[...])
        if k == 0:
            copies[7].wait()
        h = jax.nn.relu(_dot(h, v43[...]) + b43[...])
        xc = _dot(h, w44[...]) + b44[...]

    out_ref[0] = xc


def kernel(x, MM, PM, params):
    J, N, d = x.shape
    (f1w1, f1b1), (f1w2, f1b2), (f1w3, f1b3), (f1w4, f1b4) = params["f1"]
    (f2w1, f2b1), (f2w2, f2b2), (f2w3, f2b3), (f2w4, f2b4) = params["f2"]
    (f3w1, f3b1), (f3w2, f3b2), (f3w3, f3b3), (f3w4, f3b4) = params["f3"]
    (f4w1, f4b1), (f4w2, f4b2), (f4w3, f4b3), (f4w4, f4b4) = params["f4"]
    small = [f1w1, f1b1, f1b2, f1b3, f1w4, f1b4,
             f2w1, f2b1, f2b2, f2b3, f2w4, f2b4,
             f3b1, f3b2, f3b3, f3w4, f3b4,
             f4w1, f4b1, f4b2, f4b3, f4w4, f4b4]
    big = [f1w2, f2w2, f1w3, f2w3, f3w2, f3w3, f4w2, f4w3]
    vm = pltpu.MemorySpace.VMEM
    in_specs = ([pl.BlockSpec(memory_space=vm)] * (2 + len(small))
                + [pl.BlockSpec(memory_space=pltpu.MemorySpace.HBM)] * _BIG)
    scratch_shapes = ([pltpu.VMEM((256, 256), jnp.float32)] * _BIG
                      + [pltpu.SemaphoreType.DMA((_BIG,))])
    out = pl.pallas_call(
        _fused_kernel,
        out_shape=jax.ShapeDtypeStruct((J, N, d), jnp.float32),
        in_specs=in_specs,
        out_specs=pl.BlockSpec(memory_space=vm),
        scratch_shapes=scratch_shapes,
    )(x, MM, *small, *big)
    return out
